# Initial kernel scaffold; baseline (speedup 1.0000x reference)
import jax
import jax.numpy as jnp
from jax.experimental import pallas as pl


def kernel(x, pilots):
    raise NotImplementedError("write your pallas kernel here")


# speedup vs baseline: 1.0296x; 1.0296x over previous
import functools

import jax
import jax.numpy as jnp
from jax import lax
from jax.experimental import pallas as pl
from jax.experimental.pallas import tpu as pltpu
from jax.experimental.pallas import tpu_sc as plsc

NUM_TX = 4
NUM_STREAMS = 2
NUM_OFDM = 14
FFT = 4096
NUM_DATA = 12
BATCH = 64
GROUPS = NUM_TX * NUM_STREAMS

_DATA_RUNS = ((0, 0, 2), (3, 2, 8), (12, 10, 2))
_PILOT_SYMS = (2, 11)


def _grid_body(x_hbm, p_hbm, out_hbm, sem):
    info = plsc.get_sparse_core_info()
    nc, ns = info.num_cores, info.num_subcores
    nw = nc * ns
    b_per_w = BATCH // nw
    wid = lax.axis_index("s") * nc + lax.axis_index("c")
    b0 = wid * b_per_w
    r0 = b0 * GROUPS

    copies = []
    for dst0, src0, n in _DATA_RUNS:
        copies.append(
            pltpu.async_copy(
                x_hbm.at[pl.ds(r0, b_per_w * GROUPS), pl.ds(src0 * FFT, n * FFT)],
                out_hbm.at[pl.ds(r0, b_per_w * GROUPS), pl.ds(dst0 * FFT, n * FFT)],
                sem,
            )
        )
    for i in range(b_per_w):
        for pi, sym in enumerate(_PILOT_SYMS):
            copies.append(
                pltpu.async_copy(
                    p_hbm.at[:, pl.ds(pi * FFT, FFT)],
                    out_hbm.at[
                        pl.ds(r0 + i * GROUPS, GROUPS), pl.ds(sym * FFT, FFT)
                    ],
                    sem,
                )
            )
    for c in copies:
        c.wait()


def kernel(x, pilots):
    xr = x.reshape(BATCH * GROUPS, NUM_DATA * FFT)
    pr = pilots.reshape(GROUPS, 2 * FFT)
    mesh = plsc.VectorSubcoreMesh(core_axis_name="c", subcore_axis_name="s")
    run = functools.partial(
        pl.kernel,
        mesh=mesh,
        out_type=jax.ShapeDtypeStruct((BATCH * GROUPS, NUM_OFDM * FFT), jnp.float32),
        scratch_types=[pltpu.SemaphoreType.DMA],
    )(_grid_body)
    out = run(xr, pr)
    return out.reshape(BATCH, NUM_TX, NUM_STREAMS, NUM_OFDM, FFT)


# speedup vs baseline: 10.3840x; 10.0858x over previous
import functools

import jax
import jax.numpy as jnp
from jax import lax
from jax.experimental import pallas as pl
from jax.experimental.pallas import tpu as pltpu
from jax.experimental.pallas import tpu_sc as plsc

NUM_TX = 4
NUM_STREAMS = 2
NUM_OFDM = 14
FFT = 4096
NUM_DATA = 12
BATCH = 64
GROUPS_PER_BATCH = NUM_TX * NUM_STREAMS
NUM_GROUPS = BATCH * GROUPS_PER_BATCH

_SLOT_SRC = []
for _sym in range(NUM_OFDM):
    if _sym == 2:
        _SLOT_SRC.append(("p", 0))
    elif _sym == 11:
        _SLOT_SRC.append(("p", 1))
    else:
        _SLOT_SRC.append(("d", _sym - sum(1 for q in (2, 11) if q < _sym)))

_NBUF = 2


def _body(x_hbm, p_hbm, out_hbm, buf0, buf1, g0, g1, s0, s1):
    bufs = (buf0, buf1)
    gsems = (g0, g1)
    ssems = (s0, s1)

    info = plsc.get_sparse_core_info()
    nw = info.num_cores * info.num_subcores
    per_w = NUM_GROUPS // nw
    wid = lax.axis_index("s") * info.num_cores + lax.axis_index("c")
    base = wid * per_w

    def fire_gathers(i):
        g = base + i
        buf = bufs[i % _NBUF]
        sem = gsems[i % _NBUF]
        ts = lax.rem(g, GROUPS_PER_BATCH)
        cs = []
        for sym, (kind, j) in enumerate(_SLOT_SRC):
            if kind == "d":
                src = x_hbm.at[pl.ds(g * NUM_DATA * FFT + j * FFT, FFT)]
            else:
                src = p_hbm.at[pl.ds(ts * 2 * FFT + j * FFT, FFT)]
            cs.append(pltpu.async_copy(src, buf.at[sym], sem))
        return cs

    def fire_scatter(i):
        g = base + i
        b = lax.div(g, GROUPS_PER_BATCH)
        r = lax.rem(g, GROUPS_PER_BATCH)
        t = lax.div(r, NUM_STREAMS)
        s = lax.rem(r, NUM_STREAMS)
        return pltpu.async_copy(
            bufs[i % _NBUF], out_hbm.at[b, t, s], ssems[i % _NBUF]
        )

    gather_handles = [None] * per_w
    scatter_handles = [None] * per_w
    gather_handles[0] = fire_gathers(0)
    for i in range(per_w):
        for c in gather_handles[i]:
            c.wait()
        scatter_handles[i] = fire_scatter(i)
        if i + 1 < per_w:
            if i >= 1:
                scatter_handles[i - 1].wait()
            gather_handles[i + 1] = fire_gathers(i + 1)
    scatter_handles[per_w - 2].wait()
    scatter_handles[per_w - 1].wait()


def kernel(x, pilots):
    mesh = plsc.VectorSubcoreMesh(core_axis_name="c", subcore_axis_name="s")
    run = functools.partial(
        pl.kernel,
        mesh=mesh,
        out_type=jax.ShapeDtypeStruct(
            (BATCH, NUM_TX, NUM_STREAMS, NUM_OFDM, FFT), jnp.float32
        ),
        scratch_types=(
            [pltpu.VMEM((NUM_OFDM, FFT), jnp.float32)] * _NBUF
            + [pltpu.SemaphoreType.DMA] * (2 * _NBUF)
        ),
    )(_body)
    return run(x.reshape(-1), pilots.reshape(-1))


# speedup vs baseline: 35.1914x; 3.3890x over previous
import functools

import jax
import jax.numpy as jnp
from jax import lax
from jax.experimental import pallas as pl
from jax.experimental.pallas import tpu as pltpu
from jax.experimental.pallas import tpu_sc as plsc

NUM_TX = 4
NUM_STREAMS = 2
NUM_OFDM = 14
FFT = 4096
NUM_DATA = 12
BATCH = 64
LANE = 128
FB = FFT // LANE
HALF = NUM_OFDM // 2
TASKS_PER_BATCH = NUM_TX * 2
NUM_TASKS = BATCH * TASKS_PER_BATCH

_SLOT_SRC = []
for _sym in range(NUM_OFDM):
    if _sym == 2:
        _SLOT_SRC.append(("p", 0))
    elif _sym == 11:
        _SLOT_SRC.append(("p", 1))
    else:
        _SLOT_SRC.append(("d", _sym - sum(1 for q in (2, 11) if q < _sym)))

_NBUF = 2


def _body(x_hbm, p_hbm, out_hbm, buf0, buf1, g0, g1, s0, s1):
    bufs = (buf0, buf1)
    gsems = (g0, g1)
    ssems = (s0, s1)

    info = plsc.get_sparse_core_info()
    nw = info.num_cores * info.num_subcores
    per_w = NUM_TASKS // nw
    wid = lax.axis_index("s") * info.num_cores + lax.axis_index("c")
    base = wid * per_w

    def task_coords(i):
        g = base + i
        b = lax.div(g, TASKS_PER_BATCH)
        r = lax.rem(g, TASKS_PER_BATCH)
        t = lax.div(r, 2)
        h = lax.rem(r, 2)
        return b, t, h

    def fire_gathers(i, h_static):
        b, t, h = task_coords(i)
        buf = bufs[i % _NBUF]
        sem = gsems[i % _NBUF]
        cs = []
        for k in range(HALF):
            sym = h_static * HALF + k
            kind, j = _SLOT_SRC[sym]
            if kind == "d":
                src = x_hbm.at[b, t, pl.ds(j * FB, FB), :, :]
            else:
                src = p_hbm.at[t, j]
            cs.append(pltpu.async_copy(src, buf.at[k], sem))
        return cs

    def fire_scatter(i, h_static):
        b, t, h = task_coords(i)
        return pltpu.async_copy(
            bufs[i % _NBUF],
            out_hbm.at[b, t, pl.ds(h_static * HALF, HALF), :, :, :],
            ssems[i % _NBUF],
        )

    gather_handles = [None] * per_w
    scatter_handles = [None] * per_w
    gather_handles[0] = fire_gathers(0, 0)
    for i in range(per_w):
        for c in gather_handles[i]:
            c.wait()
        scatter_handles[i] = fire_scatter(i, i % 2)
        if i + 1 < per_w:
            if i >= 1:
                scatter_handles[i - 1].wait()
            gather_handles[i + 1] = fire_gathers(i + 1, (i + 1) % 2)
    scatter_handles[per_w - 2].wait()
    scatter_handles[per_w - 1].wait()


def kernel(x, pilots):
    xp = x.reshape(BATCH, NUM_TX, NUM_STREAMS, NUM_DATA * FB, LANE).transpose(
        0, 1, 3, 2, 4
    )
    pp = pilots.reshape(NUM_TX, NUM_STREAMS, 2, FB, LANE).transpose(
        0, 2, 3, 1, 4
    )

    mesh = plsc.VectorSubcoreMesh(core_axis_name="c", subcore_axis_name="s")
    run = functools.partial(
        pl.kernel,
        mesh=mesh,
        out_type=jax.ShapeDtypeStruct(
            (BATCH, NUM_TX, NUM_OFDM, FB, NUM_STREAMS, LANE), jnp.float32
        ),
        scratch_types=(
            [pltpu.VMEM((HALF, FB, NUM_STREAMS, LANE), jnp.float32)] * _NBUF
            + [pltpu.SemaphoreType.DMA] * (2 * _NBUF)
        ),
    )(_body)
    out = run(xp, pp)
    return out.transpose(0, 1, 4, 2, 3, 5).reshape(
        BATCH, NUM_TX, NUM_STREAMS, NUM_OFDM, FFT
    )


# speedup vs baseline: 36.0811x; 1.0253x over previous
import functools

import jax
import jax.numpy as jnp
from jax import lax
from jax.experimental import pallas as pl
from jax.experimental.pallas import tpu as pltpu
from jax.experimental.pallas import tpu_sc as plsc

NUM_TX = 4
NUM_STREAMS = 2
NUM_OFDM = 14
FFT = 4096
NUM_DATA = 12
BATCH = 64
LANE = 128
FB = FFT // LANE
HALF = NUM_OFDM // 2
TASKS_PER_BATCH = NUM_TX * 2
NUM_TASKS = BATCH * TASKS_PER_BATCH

_HALF_PLAN = (
    (("d", 0, 0, 2 * FB), ("p", 0, 2 * FB, FB), ("d", 2 * FB, 3 * FB, 4 * FB)),
    (("d", 6 * FB, 0, 4 * FB), ("p", 1, 4 * FB, FB), ("d", 10 * FB, 5 * FB, 2 * FB)),
)

_NBUF = 2


def _body(x_hbm, p_hbm, out_hbm, buf0, buf1, g0, g1, s0, s1):
    bufs = (buf0, buf1)
    gsems = (g0, g1)
    ssems = (s0, s1)

    info = plsc.get_sparse_core_info()
    nw = info.num_cores * info.num_subcores
    per_w = NUM_TASKS // nw
    wid = lax.axis_index("s") * info.num_cores + lax.axis_index("c")
    base = wid * per_w

    def task_coords(i):
        g = base + i
        b = lax.div(g, TASKS_PER_BATCH)
        r = lax.rem(g, TASKS_PER_BATCH)
        t = lax.div(r, 2)
        h = lax.rem(r, 2)
        return b, t, h

    def fire_gathers(i, h_static):
        b, t, h = task_coords(i)
        buf = bufs[i % _NBUF]
        sem = gsems[i % _NBUF]
        cs = []
        for kind, src0, dst0, n in _HALF_PLAN[h_static]:
            if kind == "d":
                src = x_hbm.at[b, t, pl.ds(src0, n), :, :]
            else:
                src = p_hbm.at[t, src0]
            cs.append(pltpu.async_copy(src, buf.at[pl.ds(dst0, n)], sem))
        return cs

    def fire_scatter(i, h_static):
        b, t, h = task_coords(i)
        return pltpu.async_copy(
            bufs[i % _NBUF],
            out_hbm.at[b, t, pl.ds(h_static * HALF * FB, HALF * FB), :, :],
            ssems[i % _NBUF],
        )

    gather_handles = [None] * per_w
    scatter_handles = [None] * per_w
    gather_handles[0] = fire_gathers(0, 0)
    for i in range(per_w):
        for c in gather_handles[i]:
            c.wait()
        scatter_handles[i] = fire_scatter(i, i % 2)
        if i + 1 < per_w:
            if i >= 1:
                scatter_handles[i - 1].wait()
            gather_handles[i + 1] = fire_gathers(i + 1, (i + 1) % 2)
    scatter_handles[per_w - 2].wait()
    scatter_handles[per_w - 1].wait()


def kernel(x, pilots):
    xp = x.reshape(BATCH, NUM_TX, NUM_STREAMS, NUM_DATA * FB, LANE).transpose(
        0, 1, 3, 2, 4
    )
    pp = pilots.reshape(NUM_TX, NUM_STREAMS, 2, FB, LANE).transpose(
        0, 2, 3, 1, 4
    )

    mesh = plsc.VectorSubcoreMesh(core_axis_name="c", subcore_axis_name="s")
    run = functools.partial(
        pl.kernel,
        mesh=mesh,
        out_type=jax.ShapeDtypeStruct(
            (BATCH, NUM_TX, NUM_OFDM * FB, NUM_STREAMS, LANE), jnp.float32
        ),
        scratch_types=(
            [pltpu.VMEM((HALF * FB, NUM_STREAMS, LANE), jnp.float32)] * _NBUF
            + [pltpu.SemaphoreType.DMA] * (2 * _NBUF)
        ),
    )(_body)
    out = run(xp, pp)
    return (
        out.reshape(BATCH, NUM_TX, NUM_OFDM, FB, NUM_STREAMS, LANE)
        .transpose(0, 1, 4, 2, 3, 5)
        .reshape(BATCH, NUM_TX, NUM_STREAMS, NUM_OFDM, FFT)
    )


# speedup vs baseline: 37.3412x; 1.0349x over previous
import functools

import jax
import jax.numpy as jnp
from jax import lax
from jax.experimental import pallas as pl
from jax.experimental.pallas import tpu as pltpu
from jax.experimental.pallas import tpu_sc as plsc

NUM_TX = 4
NUM_STREAMS = 2
NUM_OFDM = 14
FFT = 4096
NUM_DATA = 12
BATCH = 64
LANE = 128
FB = FFT // LANE
TASKS_PER_BATCH = NUM_TX * 4
NUM_TASKS = BATCH * TASKS_PER_BATCH

QROWS = 112
_QUARTER_PLAN = (
    (("d", 0, 0, 64), ("p", 0, 64, 32), ("d", 64, 96, 16)),
    (("d", 80, 0, 112),),
    (("d", 192, 0, 112),),
    (("d", 304, 0, 16), ("p", 1, 16, 32), ("d", 320, 48, 64)),
)

_NBUF = 4


def _body(x_hbm, p_hbm, out_hbm, buf0, buf1, buf2, buf3, g0, g1, g2, g3, s0, s1, s2, s3):
    bufs = (buf0, buf1, buf2, buf3)
    gsems = (g0, g1, g2, g3)
    ssems = (s0, s1, s2, s3)

    info = plsc.get_sparse_core_info()
    nw = info.num_cores * info.num_subcores
    per_w = NUM_TASKS // nw
    wid = lax.axis_index("s") * info.num_cores + lax.axis_index("c")
    base = wid * per_w

    def task_coords(i):
        g = base + i
        b = lax.div(g, TASKS_PER_BATCH)
        r = lax.rem(g, TASKS_PER_BATCH)
        t = lax.div(r, 4)
        return b, t

    def fire_gathers(i):
        b, t = task_coords(i)
        buf = bufs[i % _NBUF]
        sem = gsems[i % _NBUF]
        cs = []
        for kind, src0, dst0, n in _QUARTER_PLAN[i % 4]:
            if kind == "d":
                src = x_hbm.at[b, t, pl.ds(src0, n), :, :]
            else:
                src = p_hbm.at[t, src0]
            cs.append(pltpu.async_copy(src, buf.at[pl.ds(dst0, n)], sem))
        return cs

    def fire_scatter(i):
        b, t = task_coords(i)
        return pltpu.async_copy(
            bufs[i % _NBUF],
            out_hbm.at[b, t, pl.ds((i % 4) * QROWS, QROWS), :, :],
            ssems[i % _NBUF],
        )

    gather_handles = [None] * per_w
    scatter_handles = [None] * per_w
    for k in range(_NBUF - 1):
        gather_handles[k] = fire_gathers(k)
    for i in range(per_w):
        j = i + _NBUF - 1
        if j < per_w:
            if i >= 1:
                scatter_handles[i - 1].wait()
            gather_handles[j] = fire_gathers(j)
        for c in gather_handles[i]:
            c.wait()
        scatter_handles[i] = fire_scatter(i)
    for i in range(per_w - _NBUF, per_w):
        scatter_handles[i].wait()


def kernel(x, pilots):
    xp = x.reshape(BATCH, NUM_TX, NUM_STREAMS, NUM_DATA * FB, LANE).transpose(
        0, 1, 3, 2, 4
    )
    pp = pilots.reshape(NUM_TX, NUM_STREAMS, 2, FB, LANE).transpose(
        0, 2, 3, 1, 4
    )

    mesh = plsc.VectorSubcoreMesh(core_axis_name="c", subcore_axis_name="s")
    run = functools.partial(
        pl.kernel,
        mesh=mesh,
        out_type=jax.ShapeDtypeStruct(
            (BATCH, NUM_TX, NUM_OFDM * FB, NUM_STREAMS, LANE), jnp.float32
        ),
        scratch_types=(
            [pltpu.VMEM((QROWS, NUM_STREAMS, LANE), jnp.float32)] * _NBUF
            + [pltpu.SemaphoreType.DMA] * (2 * _NBUF)
        ),
    )(_body)
    out = run(xp, pp)
    return (
        out.reshape(BATCH, NUM_TX, NUM_OFDM, FB, NUM_STREAMS, LANE)
        .transpose(0, 1, 4, 2, 3, 5)
        .reshape(BATCH, NUM_TX, NUM_STREAMS, NUM_OFDM, FFT)
    )
